# Initial kernel scaffold; baseline (speedup 1.0000x reference)
import jax
import jax.numpy as jnp
from jax.experimental import pallas as pl


def kernel(inputs, W1, b1, W2, b2, W3, b3):
    raise NotImplementedError("write your pallas kernel here")


# speedup vs baseline: 2.5667x; 2.5667x over previous
import functools

import jax
import jax.numpy as jnp
from jax.experimental import pallas as pl
from jax.experimental.pallas import tpu as pltpu

FEATURES = 16
NUM_BINS = 8
TAIL = 3.0
MINW = 0.001
MINH = 0.001
MIND = 0.001
PARAM_DIM = 3 * NUM_BINS + 1


def _cumsum_sublane(a, n):
    s = a
    shift = 1
    while shift < n:
        z = jnp.zeros(s.shape[:1] + (shift,) + s.shape[2:], dtype=s.dtype)
        s = s + jnp.concatenate([z, s[:, :-shift]], axis=1)
        shift *= 2
    return s


def _rqs_kernel(xT_ref, w1_ref, b1_ref, w2_ref, b2_ref, w3_ref, b3_ref,
                out_ref, ld_ref):
    xT = xT_ref[...]
    h1 = jnp.maximum(
        jnp.dot(w1_ref[...], xT, preferred_element_type=jnp.float32)
        + b1_ref[...], 0.0)
    h2 = jnp.maximum(
        jnp.dot(w2_ref[...], h1, preferred_element_type=jnp.float32)
        + b2_ref[...], 0.0)
    p = jnp.dot(w3_ref[...], h2, preferred_element_type=jnp.float32) \
        + b3_ref[...]

    T = xT.shape[-1]
    nb = NUM_BINS
    F = FEATURES
    wl = p[0:F * nb].reshape(F, nb, T)
    hl = p[F * nb:2 * F * nb].reshape(F, nb, T)
    dl = p[2 * F * nb:].reshape(F, nb + 1, T)

    we = jnp.exp(wl - jnp.max(wl, axis=1, keepdims=True))
    w = we / jnp.sum(we, axis=1, keepdims=True)
    he = jnp.exp(hl - jnp.max(hl, axis=1, keepdims=True))
    h = he / jnp.sum(he, axis=1, keepdims=True)
    d = jnp.maximum(dl, 0.0) + jnp.log1p(jnp.exp(-jnp.abs(dl))) + MIND

    w = w * (2.0 * TAIL) * (1.0 - nb * MINW) + MINW
    h = h * (2.0 * TAIL) * (1.0 - nb * MINH) + MINH

    ccw = _cumsum_sublane(w, nb) - TAIL
    cch = _cumsum_sublane(h, nb) - TAIL

    inside = (xT >= -TAIL) & (xT <= TAIL)
    xs = jnp.where(inside, xT, 0.0)

    xs3 = xs[:, None, :]
    idx = jnp.sum((xs3 >= ccw).astype(jnp.int32), axis=1)
    idx = jnp.minimum(idx, nb - 1)

    iota8 = jax.lax.broadcasted_iota(jnp.int32, (F, nb, T), 1)
    one8 = (iota8 == idx[:, None, :]).astype(jnp.float32)
    iota9 = jax.lax.broadcasted_iota(jnp.int32, (F, nb + 1, T), 1)
    one9a = (iota9 == idx[:, None, :]).astype(jnp.float32)
    one9b = (iota9 == (idx[:, None, :] + 1)).astype(jnp.float32)

    neg_tail = jnp.full((F, 1, T), -TAIL, dtype=jnp.float32)
    cw_lo = jnp.concatenate([neg_tail, ccw[:, :nb - 1]], axis=1)
    ch_lo = jnp.concatenate([neg_tail, cch[:, :nb - 1]], axis=1)

    icw = jnp.sum(cw_lo * one8, axis=1)
    iw = jnp.sum(w * one8, axis=1)
    ich = jnp.sum(ch_lo * one8, axis=1)
    ih = jnp.sum(h * one8, axis=1)
    d0 = jnp.sum(d * one9a, axis=1)
    d1 = jnp.sum(d * one9b, axis=1)

    theta = (xs - icw) / iw
    omt = 1.0 - theta
    th2 = theta * theta
    tho = theta * omt
    numerator = ih * (d0 * th2 + d1 * tho)
    denominator = d0 * th2 + 2.0 * d1 * tho + d1 * omt * omt
    y = ich + numerator / denominator
    deriv_num = (ih / iw) * (d1 * th2 + 2.0 * d0 * tho + d0 * omt * omt)
    ld = jnp.log(deriv_num) - 2.0 * jnp.log(denominator)

    out_ref[...] = jnp.where(inside, y, xT)
    ld_ref[...] = jnp.sum(jnp.where(inside, ld, 0.0), axis=0, keepdims=True)


@functools.partial(jax.jit, static_argnames=())
def kernel(inputs, W1, b1, W2, b2, W3, b3):
    B = inputs.shape[0]
    F = FEATURES
    nb = NUM_BINS
    T = 512

    perm = (
        [f * PARAM_DIM + k for f in range(F) for k in range(nb)]
        + [f * PARAM_DIM + nb + k for f in range(F) for k in range(nb)]
        + [f * PARAM_DIM + 2 * nb + j for f in range(F) for j in range(nb + 1)]
    )
    perm = jnp.asarray(perm, dtype=jnp.int32)

    xT = inputs.T
    w1t = W1.T
    w2t = W2.T
    w3t = W3.T[perm, :]
    b1c = b1[:, None]
    b2c = b2[:, None]
    b3c = b3[perm][:, None]

    grid = (B // T,)
    outT, ld = pl.pallas_call(
        _rqs_kernel,
        grid=grid,
        in_specs=[
            pl.BlockSpec((F, T), lambda i: (0, i)),
            pl.BlockSpec(w1t.shape, lambda i: (0, 0)),
            pl.BlockSpec(b1c.shape, lambda i: (0, 0)),
            pl.BlockSpec(w2t.shape, lambda i: (0, 0)),
            pl.BlockSpec(b2c.shape, lambda i: (0, 0)),
            pl.BlockSpec(w3t.shape, lambda i: (0, 0)),
            pl.BlockSpec(b3c.shape, lambda i: (0, 0)),
        ],
        out_specs=[
            pl.BlockSpec((F, T), lambda i: (0, i)),
            pl.BlockSpec((1, T), lambda i: (0, i)),
        ],
        out_shape=[
            jax.ShapeDtypeStruct((F, B), jnp.float32),
            jax.ShapeDtypeStruct((1, B), jnp.float32),
        ],
        compiler_params=pltpu.CompilerParams(
            dimension_semantics=("arbitrary",),
        ),
    )(xT, w1t, b1c, w2t, b2c, w3t, b3c)

    return outT.T, ld[0]


# speedup vs baseline: 4.6675x; 1.8185x over previous
import functools

import jax
import jax.numpy as jnp
import numpy as np
from jax.experimental import pallas as pl
from jax.experimental.pallas import tpu as pltpu

FEATURES = 16
NUM_BINS = 8
TAIL = 3.0
MINW = 0.001
MINH = 0.001
MIND = 0.001
PARAM_DIM = 3 * NUM_BINS + 1


def _rqs_kernel(xT_ref, w1_ref, b1_ref, w2_ref, b2_ref, w3_ref, b3_ref,
                lt_ref, sb_ref, bb_ref, big_ref, out_ref, ld_ref):
    f32 = jnp.float32

    def mm(a, b):
        return jnp.dot(a, b, preferred_element_type=f32)

    xT = xT_ref[...]
    h1 = jnp.maximum(mm(w1_ref[...], xT) + b1_ref[...], 0.0)
    h2 = jnp.maximum(mm(w2_ref[...], h1) + b2_ref[...], 0.0)
    p = mm(w3_ref[...], h2) + b3_ref[...]

    T = xT.shape[-1]
    nb = NUM_BINS
    F = FEATURES
    nf = F * nb
    wl = p[0:nf]
    hl = p[nf:2 * nf]
    dl0a = p[2 * nf:3 * nf]
    dl1a = p[3 * nf:4 * nf]

    cwscale = (2.0 * TAIL) * (1.0 - nb * MINW)
    chscale = (2.0 * TAIL) * (1.0 - nb * MINH)
    wl3 = wl.reshape(F, nb, T)
    hl3 = hl.reshape(F, nb, T)
    we3 = jnp.exp(wl3 - jnp.max(wl3, axis=1, keepdims=True))
    w3d = we3 * (cwscale / jnp.sum(we3, axis=1, keepdims=True)) + MINW
    he3 = jnp.exp(hl3 - jnp.max(hl3, axis=1, keepdims=True))
    h3d = he3 * (chscale / jnp.sum(he3, axis=1, keepdims=True)) + MINH
    w = w3d.reshape(nf, T)
    h = h3d.reshape(nf, T)

    lt = lt_ref[...]
    sb = sb_ref[...]
    bb = bb_ref[...]
    big = big_ref[...]

    cw_lo = mm(lt, w) - TAIL
    ch_lo = mm(lt, h) - TAIL
    cw_up = cw_lo + w + big

    inside = (xT >= -TAIL) & (xT <= TAIL)
    xs = jnp.where(inside, xT, 0.0)
    xsb = mm(bb, xs)

    one = ((xsb >= cw_lo) & (xsb < cw_up)).astype(f32)

    icw = mm(sb, cw_lo * one)
    ich = mm(sb, ch_lo * one)
    iw = mm(sb, w * one)
    ih = mm(sb, h * one)
    g0 = mm(sb, dl0a * one)
    g1 = mm(sb, dl1a * one)

    d0 = jnp.maximum(g0, 0.0) + jnp.log1p(jnp.exp(-jnp.abs(g0))) + MIND
    d1 = jnp.maximum(g1, 0.0) + jnp.log1p(jnp.exp(-jnp.abs(g1))) + MIND

    riw = 1.0 / iw
    theta = (xs - icw) * riw
    omt = 1.0 - theta
    th2 = theta * theta
    tho = theta * omt
    om2 = omt * omt
    den = d0 * th2 + 2.0 * d1 * tho + d1 * om2
    rden = 1.0 / den
    y = ich + ih * (d0 * th2 + d1 * tho) * rden
    deriv = ih * riw * (d1 * th2 + 2.0 * d0 * tho + d0 * om2)
    ld = jnp.log(deriv * rden * rden)

    out_ref[...] = jnp.where(inside, y, xT)
    ld_ref[...] = jnp.sum(jnp.where(inside, ld, 0.0), axis=0, keepdims=True)


@functools.partial(jax.jit, static_argnames=())
def kernel(inputs, W1, b1, W2, b2, W3, b3):
    B = inputs.shape[0]
    F = FEATURES
    nb = NUM_BINS
    nf = F * nb
    T = 512

    perm = (
        [f * PARAM_DIM + k for f in range(F) for k in range(nb)]
        + [f * PARAM_DIM + nb + k for f in range(F) for k in range(nb)]
        + [f * PARAM_DIM + 2 * nb + k for f in range(F) for k in range(nb)]
        + [f * PARAM_DIM + 2 * nb + k + 1 for f in range(F) for k in range(nb)]
    )
    perm = jnp.asarray(perm, dtype=jnp.int32)

    xT = inputs.T
    w1t = W1.T
    w2t = W2.T
    w3t = W3.T[perm, :]
    b1c = b1[:, None]
    b2c = b2[:, None]
    b3c = b3[perm][:, None]

    lt_np = np.zeros((nf, nf), dtype=np.float32)
    sb_np = np.zeros((F, nf), dtype=np.float32)
    bb_np = np.zeros((nf, F), dtype=np.float32)
    big_np = np.zeros((nf, 1), dtype=np.float32)
    for f in range(F):
        for k in range(nb):
            r = f * nb + k
            sb_np[f, r] = 1.0
            bb_np[r, f] = 1.0
            for j in range(k):
                lt_np[r, f * nb + j] = 1.0
        big_np[f * nb + nb - 1, 0] = 1e30
    lt = jnp.asarray(lt_np)
    sb = jnp.asarray(sb_np)
    bb = jnp.asarray(bb_np)
    big = jnp.asarray(big_np)

    grid = (B // T,)
    const = lambda i: (0, 0)
    outT, ld = pl.pallas_call(
        _rqs_kernel,
        grid=grid,
        in_specs=[
            pl.BlockSpec((F, T), lambda i: (0, i)),
            pl.BlockSpec(w1t.shape, const),
            pl.BlockSpec(b1c.shape, const),
            pl.BlockSpec(w2t.shape, const),
            pl.BlockSpec(b2c.shape, const),
            pl.BlockSpec(w3t.shape, const),
            pl.BlockSpec(b3c.shape, const),
            pl.BlockSpec(lt.shape, const),
            pl.BlockSpec(sb.shape, const),
            pl.BlockSpec(bb.shape, const),
            pl.BlockSpec(big.shape, const),
        ],
        out_specs=[
            pl.BlockSpec((F, T), lambda i: (0, i)),
            pl.BlockSpec((1, T), lambda i: (0, i)),
        ],
        out_shape=[
            jax.ShapeDtypeStruct((F, B), jnp.float32),
            jax.ShapeDtypeStruct((1, B), jnp.float32),
        ],
        compiler_params=pltpu.CompilerParams(
            dimension_semantics=("arbitrary",),
        ),
    )(xT, w1t, b1c, w2t, b2c, w3t, b3c, lt, sb, bb, big)

    return outT.T, ld[0]


# speedup vs baseline: 6.3168x; 1.3534x over previous
import functools

import jax
import jax.numpy as jnp
import numpy as np
from jax.experimental import pallas as pl
from jax.experimental.pallas import tpu as pltpu

FEATURES = 16
NUM_BINS = 8
TAIL = 3.0
MINW = 0.001
MINH = 0.001
MIND = 0.001
PARAM_DIM = 3 * NUM_BINS + 1


def _rqs_kernel(xT_ref, w1_ref, b1_ref, w2_ref, b2_ref, w3_ref, b3_ref,
                lt_ref, sb_ref, bb_ref, big_ref, out_ref, ld_ref):
    f32 = jnp.float32

    def mm(a, b):
        return jnp.dot(a, b, preferred_element_type=f32)

    xT = xT_ref[...]
    h1 = jnp.maximum(mm(w1_ref[...], xT) + b1_ref[...], 0.0)
    h2 = jnp.maximum(mm(w2_ref[...], h1) + b2_ref[...], 0.0)
    p = mm(w3_ref[...], h2) + b3_ref[...]

    T = xT.shape[-1]
    nb = NUM_BINS
    F = FEATURES
    nf = F * nb
    wl = p[0:nf]
    hl = p[nf:2 * nf]
    dl0a = p[2 * nf:3 * nf]
    dl1a = p[3 * nf:4 * nf]

    cwscale = (2.0 * TAIL) * (1.0 - nb * MINW)
    chscale = (2.0 * TAIL) * (1.0 - nb * MINH)
    wl3 = wl.reshape(F, nb, T)
    hl3 = hl.reshape(F, nb, T)
    we3 = jnp.exp(wl3 - jnp.max(wl3, axis=1, keepdims=True))
    w3d = we3 * (cwscale / jnp.sum(we3, axis=1, keepdims=True)) + MINW
    he3 = jnp.exp(hl3 - jnp.max(hl3, axis=1, keepdims=True))
    h3d = he3 * (chscale / jnp.sum(he3, axis=1, keepdims=True)) + MINH
    w = w3d.reshape(nf, T)
    h = h3d.reshape(nf, T)

    lt = lt_ref[...]
    sb = sb_ref[...]
    bb = bb_ref[...]
    big = big_ref[...]

    cw_lo = mm(lt, w) - TAIL
    ch_lo = mm(lt, h) - TAIL
    cw_up = cw_lo + w + big

    inside = (xT >= -TAIL) & (xT <= TAIL)
    xs = jnp.where(inside, xT, 0.0)
    xsb = mm(bb, xs)

    one = ((xsb >= cw_lo) & (xsb < cw_up)).astype(f32)

    icw = mm(sb, cw_lo * one)
    ich = mm(sb, ch_lo * one)
    iw = mm(sb, w * one)
    ih = mm(sb, h * one)
    g0 = mm(sb, dl0a * one)
    g1 = mm(sb, dl1a * one)

    d0 = jnp.maximum(g0, 0.0) + jnp.log1p(jnp.exp(-jnp.abs(g0))) + MIND
    d1 = jnp.maximum(g1, 0.0) + jnp.log1p(jnp.exp(-jnp.abs(g1))) + MIND

    riw = 1.0 / iw
    theta = (xs - icw) * riw
    omt = 1.0 - theta
    th2 = theta * theta
    tho = theta * omt
    om2 = omt * omt
    den = d0 * th2 + 2.0 * d1 * tho + d1 * om2
    rden = 1.0 / den
    y = ich + ih * (d0 * th2 + d1 * tho) * rden
    deriv = ih * riw * (d1 * th2 + 2.0 * d0 * tho + d0 * om2)
    ld = jnp.log(deriv * rden * rden)

    out_ref[...] = jnp.where(inside, y, xT)
    ld_ref[...] = jnp.sum(jnp.where(inside, ld, 0.0), axis=0, keepdims=True)


@functools.partial(jax.jit, static_argnames=())
def kernel(inputs, W1, b1, W2, b2, W3, b3):
    B = inputs.shape[0]
    F = FEATURES
    nb = NUM_BINS
    nf = F * nb
    T = 1024

    perm = (
        [f * PARAM_DIM + k for f in range(F) for k in range(nb)]
        + [f * PARAM_DIM + nb + k for f in range(F) for k in range(nb)]
        + [f * PARAM_DIM + 2 * nb + k for f in range(F) for k in range(nb)]
        + [f * PARAM_DIM + 2 * nb + k + 1 for f in range(F) for k in range(nb)]
    )
    perm = jnp.asarray(perm, dtype=jnp.int32)

    xT = inputs.T
    w1t = W1.T
    w2t = W2.T
    w3t = W3.T[perm, :]
    b1c = b1[:, None]
    b2c = b2[:, None]
    b3c = b3[perm][:, None]

    lt_np = np.zeros((nf, nf), dtype=np.float32)
    sb_np = np.zeros((F, nf), dtype=np.float32)
    bb_np = np.zeros((nf, F), dtype=np.float32)
    big_np = np.zeros((nf, 1), dtype=np.float32)
    for f in range(F):
        for k in range(nb):
            r = f * nb + k
            sb_np[f, r] = 1.0
            bb_np[r, f] = 1.0
            for j in range(k):
                lt_np[r, f * nb + j] = 1.0
        big_np[f * nb + nb - 1, 0] = 1e30
    lt = jnp.asarray(lt_np)
    sb = jnp.asarray(sb_np)
    bb = jnp.asarray(bb_np)
    big = jnp.asarray(big_np)

    grid = (B // T,)
    const = lambda i: (0, 0)
    outT, ld = pl.pallas_call(
        _rqs_kernel,
        grid=grid,
        in_specs=[
            pl.BlockSpec((F, T), lambda i: (0, i)),
            pl.BlockSpec(w1t.shape, const),
            pl.BlockSpec(b1c.shape, const),
            pl.BlockSpec(w2t.shape, const),
            pl.BlockSpec(b2c.shape, const),
            pl.BlockSpec(w3t.shape, const),
            pl.BlockSpec(b3c.shape, const),
            pl.BlockSpec(lt.shape, const),
            pl.BlockSpec(sb.shape, const),
            pl.BlockSpec(bb.shape, const),
            pl.BlockSpec(big.shape, const),
        ],
        out_specs=[
            pl.BlockSpec((F, T), lambda i: (0, i)),
            pl.BlockSpec((1, T), lambda i: (0, i)),
        ],
        out_shape=[
            jax.ShapeDtypeStruct((F, B), jnp.float32),
            jax.ShapeDtypeStruct((1, B), jnp.float32),
        ],
        compiler_params=pltpu.CompilerParams(
            dimension_semantics=("arbitrary",),
        ),
    )(xT, w1t, b1c, w2t, b2c, w3t, b3c, lt, sb, bb, big)

    return outT.T, ld[0]


# speedup vs baseline: 7.3734x; 1.1673x over previous
import functools

import jax
import jax.numpy as jnp
import numpy as np
from jax.experimental import pallas as pl
from jax.experimental.pallas import tpu as pltpu

FEATURES = 16
NUM_BINS = 8
TAIL = 3.0
MINW = 0.001
MINH = 0.001
MIND = 0.001
PARAM_DIM = 3 * NUM_BINS + 1


def _rqs_kernel(xT_ref, w1_ref, b1_ref, w2_ref, b2_ref, w3_ref, b3_ref,
                lt_ref, sb_ref, bb_ref, big_ref, out_ref, ld_ref):
    f32 = jnp.float32

    def mm(a, b):
        return jnp.dot(a, b, preferred_element_type=f32)

    xT = xT_ref[...]
    h1 = jnp.maximum(mm(w1_ref[...], xT) + b1_ref[...], 0.0)
    h2 = jnp.maximum(mm(w2_ref[...], h1) + b2_ref[...], 0.0)
    p = mm(w3_ref[...], h2) + b3_ref[...]

    T = xT.shape[-1]
    nb = NUM_BINS
    F = FEATURES
    nf = F * nb
    wl = p[0:nf]
    hl = p[nf:2 * nf]
    dl0a = p[2 * nf:3 * nf]
    dl1a = p[3 * nf:4 * nf]

    cwscale = (2.0 * TAIL) * (1.0 - nb * MINW)
    chscale = (2.0 * TAIL) * (1.0 - nb * MINH)
    wl3 = wl.reshape(F, nb, T)
    hl3 = hl.reshape(F, nb, T)
    we3 = jnp.exp(wl3 - jnp.max(wl3, axis=1, keepdims=True))
    w3d = we3 * (cwscale / jnp.sum(we3, axis=1, keepdims=True)) + MINW
    he3 = jnp.exp(hl3 - jnp.max(hl3, axis=1, keepdims=True))
    h3d = he3 * (chscale / jnp.sum(he3, axis=1, keepdims=True)) + MINH
    w = w3d.reshape(nf, T)
    h = h3d.reshape(nf, T)

    lt = lt_ref[...]
    sb = sb_ref[...]
    bb = bb_ref[...]
    big = big_ref[...]

    cw_lo = mm(lt, w) - TAIL
    ch_lo = mm(lt, h) - TAIL
    cw_up = cw_lo + w + big

    inside = (xT >= -TAIL) & (xT <= TAIL)
    xs = jnp.where(inside, xT, 0.0)
    xsb = mm(bb, xs)

    one = ((xsb >= cw_lo) & (xsb < cw_up)).astype(f32)

    icw = mm(sb, cw_lo * one)
    ich = mm(sb, ch_lo * one)
    iw = mm(sb, w * one)
    ih = mm(sb, h * one)
    g0 = mm(sb, dl0a * one)
    g1 = mm(sb, dl1a * one)

    d0 = jnp.maximum(g0, 0.0) + jnp.log1p(jnp.exp(-jnp.abs(g0))) + MIND
    d1 = jnp.maximum(g1, 0.0) + jnp.log1p(jnp.exp(-jnp.abs(g1))) + MIND

    riw = 1.0 / iw
    theta = (xs - icw) * riw
    omt = 1.0 - theta
    th2 = theta * theta
    tho = theta * omt
    om2 = omt * omt
    den = d0 * th2 + 2.0 * d1 * tho + d1 * om2
    rden = 1.0 / den
    y = ich + ih * (d0 * th2 + d1 * tho) * rden
    deriv = ih * riw * (d1 * th2 + 2.0 * d0 * tho + d0 * om2)
    ld = jnp.log(deriv * rden * rden)

    out_ref[...] = jnp.where(inside, y, xT)
    ld_ref[...] = jnp.sum(jnp.where(inside, ld, 0.0), axis=0, keepdims=True)


@functools.partial(jax.jit, static_argnames=())
def kernel(inputs, W1, b1, W2, b2, W3, b3):
    B = inputs.shape[0]
    F = FEATURES
    nb = NUM_BINS
    nf = F * nb
    T = 2048

    perm = (
        [f * PARAM_DIM + k for f in range(F) for k in range(nb)]
        + [f * PARAM_DIM + nb + k for f in range(F) for k in range(nb)]
        + [f * PARAM_DIM + 2 * nb + k for f in range(F) for k in range(nb)]
        + [f * PARAM_DIM + 2 * nb + k + 1 for f in range(F) for k in range(nb)]
    )
    perm = jnp.asarray(perm, dtype=jnp.int32)

    xT = inputs.T
    w1t = W1.T
    w2t = W2.T
    w3t = W3.T[perm, :]
    b1c = b1[:, None]
    b2c = b2[:, None]
    b3c = b3[perm][:, None]

    lt_np = np.zeros((nf, nf), dtype=np.float32)
    sb_np = np.zeros((F, nf), dtype=np.float32)
    bb_np = np.zeros((nf, F), dtype=np.float32)
    big_np = np.zeros((nf, 1), dtype=np.float32)
    for f in range(F):
        for k in range(nb):
            r = f * nb + k
            sb_np[f, r] = 1.0
            bb_np[r, f] = 1.0
            for j in range(k):
                lt_np[r, f * nb + j] = 1.0
        big_np[f * nb + nb - 1, 0] = 1e30
    lt = jnp.asarray(lt_np)
    sb = jnp.asarray(sb_np)
    bb = jnp.asarray(bb_np)
    big = jnp.asarray(big_np)

    grid = (B // T,)
    const = lambda i: (0, 0)
    outT, ld = pl.pallas_call(
        _rqs_kernel,
        grid=grid,
        in_specs=[
            pl.BlockSpec((F, T), lambda i: (0, i)),
            pl.BlockSpec(w1t.shape, const),
            pl.BlockSpec(b1c.shape, const),
            pl.BlockSpec(w2t.shape, const),
            pl.BlockSpec(b2c.shape, const),
            pl.BlockSpec(w3t.shape, const),
            pl.BlockSpec(b3c.shape, const),
            pl.BlockSpec(lt.shape, const),
            pl.BlockSpec(sb.shape, const),
            pl.BlockSpec(bb.shape, const),
            pl.BlockSpec(big.shape, const),
        ],
        out_specs=[
            pl.BlockSpec((F, T), lambda i: (0, i)),
            pl.BlockSpec((1, T), lambda i: (0, i)),
        ],
        out_shape=[
            jax.ShapeDtypeStruct((F, B), jnp.float32),
            jax.ShapeDtypeStruct((1, B), jnp.float32),
        ],
        compiler_params=pltpu.CompilerParams(
            dimension_semantics=("arbitrary",),
        ),
    )(xT, w1t, b1c, w2t, b2c, w3t, b3c, lt, sb, bb, big)

    return outT.T, ld[0]
